# Initial kernel scaffold; baseline (speedup 1.0000x reference)
import jax
import jax.numpy as jnp
from jax.experimental import pallas as pl


def kernel(hidden_states, hash_seeds):
    raise NotImplementedError("write your pallas kernel here")


# speedup vs baseline: 11.2864x; 11.2864x over previous
import functools

import jax
import jax.numpy as jnp
from jax import lax
from jax.experimental import pallas as pl
from jax.experimental.pallas import tpu as pltpu
from jax.experimental.pallas import tpu_sc as plsc

N_TOK = 32768
HID = 1024
DIMQ = 64
N_EXP = 64
NH = 4
L = 16
NC, NS = 2, 16
NW = NC * NS
TPW = N_TOK // NW
GROUPS = TPW // L


def _router_body(x_hbm, seeds_hbm, sel_hbm, mask_hbm,
                 inbuf, xbuf, selbuf, seedbuf, mb0, mb1, sem0, sem1):
    wid = lax.axis_index("s") * jnp.int32(NC) + lax.axis_index("c")
    base = wid * jnp.int32(TPW)

    pltpu.sync_copy(x_hbm.at[pl.ds(base, TPW), pl.ds(0, DIMQ)], inbuf)
    pltpu.sync_copy(seeds_hbm, seedbuf)

    iota = lax.iota(jnp.int32, L)
    zerof = jnp.full((L,), 0.0, jnp.float32)
    sevenf = jnp.full((L,), 7.0, jnp.float32)
    twelve = jnp.full((L,), 12, jnp.int32)
    four = jnp.full((L,), 4, jnp.int32)
    zeroi = jnp.full((L,), 0, jnp.int32)
    half = jnp.full((L,), 0.5, jnp.float32)
    c63 = jnp.full((L,), 63, jnp.int32)
    weights = [iota + jnp.int32(16 * g + 1) for g in range(4)]
    seeds = [seedbuf[h, :] for h in range(NH)]

    mbufs = (mb0, mb1)
    sems = (sem0, sem1)

    def group(it, b):
        g = it * jnp.int32(2) + jnp.int32(b)
        mb = mbufs[b]
        sem = sems[b]

        @pl.when(it > 0)
        def _wait():
            pltpu.make_async_copy(mb, mask_hbm.at[pl.ds(base, L)], sem).wait()

        for r in range(L):
            for c in range(4):
                mb[r, pl.ds(16 * c, 16)] = zerof

        for i in range(L):
            tok = g * jnp.int32(L) + jnp.int32(i)
            acc = None
            for gg in range(4):
                xg = inbuf[tok, pl.ds(16 * gg, 16)]
                mag = jnp.minimum(jnp.abs(xg), sevenf).astype(jnp.int32)
                s2 = jnp.where(xg < zerof, twelve,
                               jnp.where(xg > zerof, four, zeroi))
                term = (s2 | mag) * weights[gg]
                acc = term if acc is None else acc ^ term
            xbuf[i, pl.ds(0, 16)] = acc

        rk = None
        for j in range(L):
            col = plsc.load_gather(xbuf, [iota, jnp.full((L,), j, jnp.int32)])
            rk = col if rk is None else rk ^ col

        e = [(rk ^ seeds[h]) & c63 for h in range(NH)]
        a = jnp.minimum(e[0], e[1])
        bb = jnp.maximum(e[0], e[1])
        c = jnp.minimum(e[2], e[3])
        d = jnp.maximum(e[2], e[3])
        sel0 = jnp.minimum(a, c)
        sel1 = jnp.minimum(jnp.maximum(a, c), jnp.minimum(bb, d))

        idx0 = (g * jnp.int32(L) + iota) * jnp.int32(2)
        plsc.store_scatter(selbuf, [idx0], sel0)
        plsc.store_scatter(selbuf, [idx0 + 1], sel1)

        plsc.store_scatter(mb, [iota, sel0], half)
        plsc.store_scatter(mb, [iota, sel1], half)

        pltpu.async_copy(mb, mask_hbm.at[pl.ds(base + g * jnp.int32(L), L)], sem)

    def body(_, it):
        group(it, 0)
        group(it, 1)
        return it + jnp.int32(1)

    lax.fori_loop(0, GROUPS // 2, body, jnp.int32(0))

    pltpu.make_async_copy(mb0, mask_hbm.at[pl.ds(base, L)], sem0).wait()
    pltpu.make_async_copy(mb1, mask_hbm.at[pl.ds(base, L)], sem1).wait()
    pltpu.sync_copy(selbuf, sel_hbm.at[pl.ds(wid * jnp.int32(2 * TPW), 2 * TPW)])


_router = functools.partial(
    pl.kernel,
    out_type=[
        jax.ShapeDtypeStruct((N_TOK * 2,), jnp.int32),
        jax.ShapeDtypeStruct((N_TOK, N_EXP), jnp.float32),
    ],
    mesh=plsc.VectorSubcoreMesh(core_axis_name="c", subcore_axis_name="s"),
    compiler_params=pltpu.CompilerParams(
        use_tc_tiling_on_sc=False, needs_layout_passes=False),
    scratch_types=[
        pltpu.VMEM((TPW, DIMQ), jnp.float32),
        pltpu.VMEM((L, 17), jnp.int32),
        pltpu.VMEM((2 * TPW,), jnp.int32),
        pltpu.VMEM((NH, L), jnp.int32),
        pltpu.VMEM((L, N_EXP), jnp.float32),
        pltpu.VMEM((L, N_EXP), jnp.float32),
        pltpu.SemaphoreType.DMA,
        pltpu.SemaphoreType.DMA,
    ],
)(_router_body)


@jax.jit
def kernel(hidden_states, hash_seeds):
    seeds_b = jnp.broadcast_to(
        hash_seeds.astype(jnp.int32)[:, None], (NH, L))
    sel_flat, masks = _router(hidden_states, seeds_b)
    selected = sel_flat.reshape(N_TOK, 2).astype(jnp.int64)
    expert_weights = jnp.full((N_TOK, 2), 0.5, dtype=jnp.float32)
    return selected, expert_weights, masks


# speedup vs baseline: 13.3683x; 1.1845x over previous
import functools

import jax
import jax.numpy as jnp
from jax import lax
from jax.experimental import pallas as pl
from jax.experimental.pallas import tpu as pltpu
from jax.experimental.pallas import tpu_sc as plsc

N_TOK = 32768
HID = 1024
DIMQ = 64
N_EXP = 64
NH = 4
L = 16
NC, NS = 2, 16
NW = NC * NS
TPW = N_TOK // NW
GROUPS = TPW // L


def _router_body(x_hbm, seeds_hbm, sel_hbm, mask_hbm,
                 inbuf, xbuf, selbuf, seedbuf, mb0, mb1, sem0, sem1):
    wid = lax.axis_index("s") * jnp.int32(NC) + lax.axis_index("c")
    base = wid * jnp.int32(TPW)

    pltpu.sync_copy(x_hbm.at[pl.ds(base, TPW)], inbuf)
    pltpu.sync_copy(seeds_hbm, seedbuf)

    iota = lax.iota(jnp.int32, L)
    zerof = jnp.full((L,), 0.0, jnp.float32)
    sevenf = jnp.full((L,), 7.0, jnp.float32)
    twelve = jnp.full((L,), 12, jnp.int32)
    four = jnp.full((L,), 4, jnp.int32)
    zeroi = jnp.full((L,), 0, jnp.int32)
    half = jnp.full((L,), 0.5, jnp.float32)
    c63 = jnp.full((L,), 63, jnp.int32)
    weights = [iota + jnp.int32(16 * g + 1) for g in range(4)]
    seeds = [seedbuf[h, :] for h in range(NH)]

    mbufs = (mb0, mb1)
    sems = (sem0, sem1)

    def group(it, b):
        g = it * jnp.int32(2) + jnp.int32(b)
        mb = mbufs[b]
        sem = sems[b]

        @pl.when(it > 0)
        def _wait():
            pltpu.make_async_copy(mb, mask_hbm.at[pl.ds(base, L)], sem).wait()

        for r in range(L):
            for c in range(4):
                mb[r, pl.ds(16 * c, 16)] = zerof

        for i in range(L):
            tok = g * jnp.int32(L) + jnp.int32(i)
            acc = None
            for gg in range(4):
                xg = inbuf[tok, pl.ds(16 * gg, 16)]
                mag = jnp.minimum(jnp.abs(xg), sevenf).astype(jnp.int32)
                s2 = jnp.where(xg < zerof, twelve,
                               jnp.where(xg > zerof, four, zeroi))
                term = (s2 | mag) * weights[gg]
                acc = term if acc is None else acc ^ term
            xbuf[i, pl.ds(0, 16)] = acc

        rk = None
        for j in range(L):
            col = plsc.load_gather(xbuf, [iota, jnp.full((L,), j, jnp.int32)])
            rk = col if rk is None else rk ^ col

        e = [(rk ^ seeds[h]) & c63 for h in range(NH)]
        a = jnp.minimum(e[0], e[1])
        bb = jnp.maximum(e[0], e[1])
        c = jnp.minimum(e[2], e[3])
        d = jnp.maximum(e[2], e[3])
        sel0 = jnp.minimum(a, c)
        sel1 = jnp.minimum(jnp.maximum(a, c), jnp.minimum(bb, d))

        idx0 = (g * jnp.int32(L) + iota) * jnp.int32(2)
        plsc.store_scatter(selbuf, [idx0], sel0)
        plsc.store_scatter(selbuf, [idx0 + 1], sel1)

        plsc.store_scatter(mb, [iota, sel0], half)
        plsc.store_scatter(mb, [iota, sel1], half)

        pltpu.async_copy(mb, mask_hbm.at[pl.ds(base + g * jnp.int32(L), L)], sem)

    def body(_, it):
        group(it, 0)
        group(it, 1)
        return it + jnp.int32(1)

    lax.fori_loop(0, GROUPS // 2, body, jnp.int32(0))

    pltpu.make_async_copy(mb0, mask_hbm.at[pl.ds(base, L)], sem0).wait()
    pltpu.make_async_copy(mb1, mask_hbm.at[pl.ds(base, L)], sem1).wait()
    pltpu.sync_copy(selbuf, sel_hbm.at[pl.ds(wid * jnp.int32(2 * TPW), 2 * TPW)])


_router = functools.partial(
    pl.kernel,
    out_type=[
        jax.ShapeDtypeStruct((N_TOK * 2,), jnp.int32),
        jax.ShapeDtypeStruct((N_TOK, N_EXP), jnp.float32),
    ],
    mesh=plsc.VectorSubcoreMesh(core_axis_name="c", subcore_axis_name="s"),
    compiler_params=pltpu.CompilerParams(
        use_tc_tiling_on_sc=False, needs_layout_passes=False),
    scratch_types=[
        pltpu.VMEM((TPW, DIMQ), jnp.float32),
        pltpu.VMEM((L, 17), jnp.int32),
        pltpu.VMEM((2 * TPW,), jnp.int32),
        pltpu.VMEM((NH, L), jnp.int32),
        pltpu.VMEM((L, N_EXP), jnp.float32),
        pltpu.VMEM((L, N_EXP), jnp.float32),
        pltpu.SemaphoreType.DMA,
        pltpu.SemaphoreType.DMA,
    ],
)(_router_body)


@jax.jit
def kernel(hidden_states, hash_seeds):
    xs = hidden_states[:, :DIMQ]
    seeds_b = jnp.broadcast_to(
        hash_seeds.astype(jnp.int32)[:, None], (NH, L))
    sel_flat, masks = _router(xs, seeds_b)
    selected = sel_flat.reshape(N_TOK, 2).astype(jnp.int64)
    expert_weights = jnp.full((N_TOK, 2), 0.5, dtype=jnp.float32)
    return selected, expert_weights, masks


# speedup vs baseline: 56.7295x; 4.2436x over previous
import functools

import jax
import jax.numpy as jnp
from jax import lax
from jax.experimental import pallas as pl
from jax.experimental.pallas import tpu as pltpu
from jax.experimental.pallas import tpu_sc as plsc

N_TOK = 32768
HID = 1024
DIMQ = 64
N_EXP = 64
NH = 4
L = 16
NC, NS = 2, 16
NW = NC * NS
TPW = N_TOK // NW
GROUPS = TPW // L


def _router_body(x_hbm, seeds_hbm, sel_hbm, mask_hbm,
                 inbuf, xbuf, selbuf, seedbuf, mb0, mb1, sem0, sem1):
    wid = lax.axis_index("s") * jnp.int32(NC) + lax.axis_index("c")
    base = wid * jnp.int32(TPW)

    pltpu.sync_copy(x_hbm.at[pl.ds(base, TPW)], inbuf)
    pltpu.sync_copy(seeds_hbm, seedbuf)

    iota = lax.iota(jnp.int32, L)
    zerof = jnp.full((L,), 0.0, jnp.float32)
    sevenf = jnp.full((L,), 7.0, jnp.float32)
    twelve = jnp.full((L,), 12, jnp.int32)
    four = jnp.full((L,), 4, jnp.int32)
    zeroi = jnp.full((L,), 0, jnp.int32)
    half = jnp.full((L,), 0.5, jnp.float32)
    c63 = jnp.full((L,), 63, jnp.int32)
    weights = [iota + jnp.int32(16 * g + 1) for g in range(4)]
    seeds = [seedbuf[h, :] for h in range(NH)]

    mbufs = (mb0, mb1)
    sems = (sem0, sem1)

    def group(it, b):
        g = it * jnp.int32(2) + jnp.int32(b)
        mb = mbufs[b]
        sem = sems[b]

        @pl.when(it > 0)
        def _wait():
            pltpu.make_async_copy(mb, mask_hbm.at[pl.ds(base, L)], sem).wait()

        for r in range(L):
            for c in range(4):
                mb[r, pl.ds(16 * c, 16)] = zerof

        for i in range(L):
            tok = g * jnp.int32(L) + jnp.int32(i)
            acc = None
            for gg in range(4):
                xg = inbuf[tok, pl.ds(16 * gg, 16)]
                mag = jnp.minimum(jnp.abs(xg), sevenf).astype(jnp.int32)
                s2 = jnp.where(xg < zerof, twelve,
                               jnp.where(xg > zerof, four, zeroi))
                term = (s2 | mag) * weights[gg]
                acc = term if acc is None else acc ^ term
            xbuf[i, pl.ds(0, 16)] = acc

        rk = None
        for j in range(L):
            col = plsc.load_gather(xbuf, [iota, jnp.full((L,), j, jnp.int32)])
            rk = col if rk is None else rk ^ col

        e = [(rk ^ seeds[h]) & c63 for h in range(NH)]
        a = jnp.minimum(e[0], e[1])
        bb = jnp.maximum(e[0], e[1])
        c = jnp.minimum(e[2], e[3])
        d = jnp.maximum(e[2], e[3])
        sel0 = jnp.minimum(a, c)
        sel1 = jnp.minimum(jnp.maximum(a, c), jnp.minimum(bb, d))

        gl = g * jnp.int32(L)
        selbuf[0, pl.ds(gl, L)] = sel0
        selbuf[1, pl.ds(gl, L)] = sel1

        plsc.store_scatter(mb, [iota, sel0], half)
        plsc.store_scatter(mb, [iota, sel1], half)

        pltpu.async_copy(mb, mask_hbm.at[pl.ds(base + g * jnp.int32(L), L)], sem)

    def body(_, it):
        group(it, 0)
        group(it, 1)
        return it + jnp.int32(1)

    lax.fori_loop(0, GROUPS // 2, body, jnp.int32(0))

    pltpu.make_async_copy(mb0, mask_hbm.at[pl.ds(base, L)], sem0).wait()
    pltpu.make_async_copy(mb1, mask_hbm.at[pl.ds(base, L)], sem1).wait()
    pltpu.sync_copy(selbuf, sel_hbm.at[:, pl.ds(base, TPW)])


_router = functools.partial(
    pl.kernel,
    out_type=[
        jax.ShapeDtypeStruct((2, N_TOK), jnp.int32),
        jax.ShapeDtypeStruct((N_TOK, N_EXP), jnp.float32),
    ],
    mesh=plsc.VectorSubcoreMesh(core_axis_name="c", subcore_axis_name="s"),
    compiler_params=pltpu.CompilerParams(
        use_tc_tiling_on_sc=False, needs_layout_passes=False),
    scratch_types=[
        pltpu.VMEM((TPW, DIMQ), jnp.float32),
        pltpu.VMEM((L, 17), jnp.int32),
        pltpu.VMEM((2, TPW), jnp.int32),
        pltpu.VMEM((NH, L), jnp.int32),
        pltpu.VMEM((L, N_EXP), jnp.float32),
        pltpu.VMEM((L, N_EXP), jnp.float32),
        pltpu.SemaphoreType.DMA,
        pltpu.SemaphoreType.DMA,
    ],
)(_router_body)


@jax.jit
def kernel(hidden_states, hash_seeds):
    xs = hidden_states[:, :DIMQ]
    seeds_b = jnp.broadcast_to(
        hash_seeds.astype(jnp.int32)[:, None], (NH, L))
    sel_planar, masks = _router(xs, seeds_b)
    selected = sel_planar.astype(jnp.int64).T
    expert_weights = jnp.full((N_TOK, 2), 0.5, dtype=jnp.float32)
    return selected, expert_weights, masks


# speedup vs baseline: 65.2991x; 1.1511x over previous
import functools

import jax
import jax.numpy as jnp
from jax import lax
from jax.experimental import pallas as pl
from jax.experimental.pallas import tpu as pltpu
from jax.experimental.pallas import tpu_sc as plsc

N_TOK = 32768
HID = 1024
DIMQ = 64
N_EXP = 64
NH = 4
L = 16
NC, NS = 2, 16
NW = NC * NS
TPW = N_TOK // NW
GROUPS = TPW // L


def _router_body(x_hbm, seeds_hbm, sel_hbm, mask_hbm,
                 inbuf, xbuf, selbuf, seedbuf, mb0, mb1, sem0, sem1):
    wid = lax.axis_index("s") * jnp.int32(NC) + lax.axis_index("c")
    base = wid * jnp.int32(TPW)

    pltpu.sync_copy(x_hbm.at[pl.ds(base, TPW)], inbuf)
    pltpu.sync_copy(seeds_hbm, seedbuf)

    iota = lax.iota(jnp.int32, L)
    zerof = jnp.full((L,), 0.0, jnp.float32)
    sevenf = jnp.full((L,), 7.0, jnp.float32)
    twelve = jnp.full((L,), 12, jnp.int32)
    four = jnp.full((L,), 4, jnp.int32)
    zeroi = jnp.full((L,), 0, jnp.int32)
    half = jnp.full((L,), 0.5, jnp.float32)
    c63 = jnp.full((L,), 63, jnp.int32)
    weights = [iota + jnp.int32(16 * g + 1) for g in range(4)]
    seeds = [seedbuf[h, :] for h in range(NH)]

    mbufs = (mb0, mb1)
    sems = (sem0, sem1)

    for mb in mbufs:
        for r in range(L):
            for c in range(4, 8):
                mb[r, pl.ds(16 * c, 16)] = zerof

    def group(it, b):
        g = it * jnp.int32(2) + jnp.int32(b)
        mb = mbufs[b]
        sem = sems[b]

        @pl.when(it > 0)
        def _wait():
            pltpu.make_async_copy(mb, mask_hbm.at[pl.ds(base, L)], sem).wait()

        for r in range(L):
            for c in range(4):
                mb[r, pl.ds(16 * c, 16)] = zerof

        for i in range(L):
            tok = g * jnp.int32(L) + jnp.int32(i)
            acc = None
            for gg in range(4):
                xg = inbuf[tok, pl.ds(16 * gg, 16)]
                mag = jnp.minimum(jnp.abs(xg), sevenf).astype(jnp.int32)
                s2 = jnp.where(xg < zerof, twelve,
                               jnp.where(xg > zerof, four, zeroi))
                term = (s2 | mag) * weights[gg]
                acc = term if acc is None else acc ^ term
            xbuf[i, pl.ds(0, 16)] = acc

        rk = None
        for j in range(L):
            col = plsc.load_gather(xbuf, [iota, jnp.full((L,), j, jnp.int32)])
            rk = col if rk is None else rk ^ col

        e = [(rk ^ seeds[h]) & c63 for h in range(NH)]
        a = jnp.minimum(e[0], e[1])
        bb = jnp.maximum(e[0], e[1])
        c = jnp.minimum(e[2], e[3])
        d = jnp.maximum(e[2], e[3])
        sel0 = jnp.minimum(a, c)
        sel1 = jnp.minimum(jnp.maximum(a, c), jnp.minimum(bb, d))

        gl = g * jnp.int32(L)
        selbuf[0, pl.ds(gl, L)] = sel0
        selbuf[1, pl.ds(gl, L)] = sel1

        plsc.store_scatter(mb, [iota, sel0], half)
        plsc.store_scatter(mb, [iota, sel1], half)

        pltpu.async_copy(mb, mask_hbm.at[pl.ds(base + g * jnp.int32(L), L)], sem)

    def body(_, it):
        group(it, 0)
        group(it, 1)
        return it + jnp.int32(1)

    lax.fori_loop(0, GROUPS // 2, body, jnp.int32(0))

    pltpu.make_async_copy(mb0, mask_hbm.at[pl.ds(base, L)], sem0).wait()
    pltpu.make_async_copy(mb1, mask_hbm.at[pl.ds(base, L)], sem1).wait()
    pltpu.sync_copy(selbuf, sel_hbm.at[:, pl.ds(base, TPW)])


_router = functools.partial(
    pl.kernel,
    out_type=[
        jax.ShapeDtypeStruct((2, N_TOK), jnp.int32),
        jax.ShapeDtypeStruct((N_TOK, 128), jnp.float32),
    ],
    mesh=plsc.VectorSubcoreMesh(core_axis_name="c", subcore_axis_name="s"),
    compiler_params=pltpu.CompilerParams(
        use_tc_tiling_on_sc=False, needs_layout_passes=False),
    scratch_types=[
        pltpu.VMEM((TPW, DIMQ), jnp.float32),
        pltpu.VMEM((L, 17), jnp.int32),
        pltpu.VMEM((2, TPW), jnp.int32),
        pltpu.VMEM((NH, L), jnp.int32),
        pltpu.VMEM((L, 128), jnp.float32),
        pltpu.VMEM((L, 128), jnp.float32),
        pltpu.SemaphoreType.DMA,
        pltpu.SemaphoreType.DMA,
    ],
)(_router_body)


@jax.jit
def kernel(hidden_states, hash_seeds):
    xs = hidden_states[:, :DIMQ]
    seeds_b = jnp.broadcast_to(
        hash_seeds.astype(jnp.int32)[:, None], (NH, L))
    sel_planar, masks_wide = _router(xs, seeds_b)
    masks = masks_wide[:, :N_EXP]
    selected = sel_planar.astype(jnp.int64).T
    expert_weights = jnp.full((N_TOK, 2), 0.5, dtype=jnp.float32)
    return selected, expert_weights, masks


# speedup vs baseline: 78.2622x; 1.1985x over previous
import functools

import jax
import jax.numpy as jnp
from jax import lax
from jax.experimental import pallas as pl
from jax.experimental.pallas import tpu as pltpu
from jax.experimental.pallas import tpu_sc as plsc

N_TOK = 32768
HID = 1024
DIMQ = 64
SLICE_W = 128
N_EXP = 64
NH = 4
L = 16
NC, NS = 2, 16
NW = NC * NS
TPW = N_TOK // NW
GROUPS = TPW // L


def _router_body(x_hbm, seeds_hbm, sel_hbm, mask_hbm,
                 inbuf, xbuf, selbuf, seedbuf, mb0, mb1, sem0, sem1):
    wid = lax.axis_index("s") * jnp.int32(NC) + lax.axis_index("c")
    base = wid * jnp.int32(TPW)

    pltpu.sync_copy(x_hbm.at[pl.ds(base, TPW), pl.ds(0, DIMQ)], inbuf)
    pltpu.sync_copy(seeds_hbm, seedbuf)

    iota = lax.iota(jnp.int32, L)
    zerof = jnp.full((L,), 0.0, jnp.float32)
    sevenf = jnp.full((L,), 7.0, jnp.float32)
    twelve = jnp.full((L,), 12, jnp.int32)
    four = jnp.full((L,), 4, jnp.int32)
    zeroi = jnp.full((L,), 0, jnp.int32)
    half = jnp.full((L,), 0.5, jnp.float32)
    c63 = jnp.full((L,), 63, jnp.int32)
    weights = [iota + jnp.int32(16 * g + 1) for g in range(4)]
    seeds = [seedbuf[h, :] for h in range(NH)]

    mbufs = (mb0, mb1)
    sems = (sem0, sem1)

    def group(it, b):
        g = it * jnp.int32(2) + jnp.int32(b)
        mb = mbufs[b]
        sem = sems[b]
        gl = g * jnp.int32(L)

        @pl.when(it > 0)
        def _wait():
            pltpu.make_async_copy(mb, mask_hbm.at[:, pl.ds(base, L)], sem).wait()

        for r in range(N_EXP):
            mb[r, pl.ds(0, L)] = zerof

        for i in range(L):
            tok = gl + jnp.int32(i)
            acc = None
            for gg in range(4):
                xg = inbuf[tok, pl.ds(16 * gg, 16)]
                mag = jnp.minimum(jnp.abs(xg), sevenf).astype(jnp.int32)
                s2 = jnp.where(xg < zerof, twelve,
                               jnp.where(xg > zerof, four, zeroi))
                term = (s2 | mag) * weights[gg]
                acc = term if acc is None else acc ^ term
            xbuf[i, pl.ds(0, 16)] = acc

        rk = None
        for j in range(L):
            col = plsc.load_gather(xbuf, [iota, jnp.full((L,), j, jnp.int32)])
            rk = col if rk is None else rk ^ col

        e = [(rk ^ seeds[h]) & c63 for h in range(NH)]
        a = jnp.minimum(e[0], e[1])
        bb = jnp.maximum(e[0], e[1])
        c = jnp.minimum(e[2], e[3])
        d = jnp.maximum(e[2], e[3])
        sel0 = jnp.minimum(a, c)
        sel1 = jnp.minimum(jnp.maximum(a, c), jnp.minimum(bb, d))

        selbuf[0, pl.ds(gl, L)] = sel0
        selbuf[1, pl.ds(gl, L)] = sel1

        plsc.store_scatter(mb, [sel0, iota], half)
        plsc.store_scatter(mb, [sel1, iota], half)

        pltpu.async_copy(mb, mask_hbm.at[:, pl.ds(base + gl, L)], sem)

    def body(_, it):
        group(it, 0)
        group(it, 1)
        return it + jnp.int32(1)

    lax.fori_loop(0, GROUPS // 2, body, jnp.int32(0))

    pltpu.make_async_copy(mb0, mask_hbm.at[:, pl.ds(base, L)], sem0).wait()
    pltpu.make_async_copy(mb1, mask_hbm.at[:, pl.ds(base, L)], sem1).wait()
    pltpu.sync_copy(selbuf, sel_hbm.at[:, pl.ds(base, TPW)])


_router = functools.partial(
    pl.kernel,
    out_type=[
        jax.ShapeDtypeStruct((2, N_TOK), jnp.int32),
        jax.ShapeDtypeStruct((N_EXP, N_TOK), jnp.float32),
    ],
    mesh=plsc.VectorSubcoreMesh(core_axis_name="c", subcore_axis_name="s"),
    compiler_params=pltpu.CompilerParams(
        use_tc_tiling_on_sc=False, needs_layout_passes=False),
    scratch_types=[
        pltpu.VMEM((TPW, DIMQ), jnp.float32),
        pltpu.VMEM((L, 17), jnp.int32),
        pltpu.VMEM((2, TPW), jnp.int32),
        pltpu.VMEM((NH, L), jnp.int32),
        pltpu.VMEM((N_EXP, L), jnp.float32),
        pltpu.VMEM((N_EXP, L), jnp.float32),
        pltpu.SemaphoreType.DMA,
        pltpu.SemaphoreType.DMA,
    ],
)(_router_body)


@jax.jit
def kernel(hidden_states, hash_seeds):
    xs = hidden_states[:, :SLICE_W]
    seeds_b = jnp.broadcast_to(
        hash_seeds.astype(jnp.int32)[:, None], (NH, L))
    sel_planar, masks_t = _router(xs, seeds_b)
    masks = masks_t.T
    selected = sel_planar.astype(jnp.int64).T
    expert_weights = jnp.full((N_TOK, 2), 0.5, dtype=jnp.float32)
    return selected, expert_weights, masks


# speedup vs baseline: 89.8236x; 1.1477x over previous
import functools

import jax
import jax.numpy as jnp
from jax import lax
from jax.experimental import pallas as pl
from jax.experimental.pallas import tpu as pltpu
from jax.experimental.pallas import tpu_sc as plsc

N_TOK = 32768
HID = 1024
DIMQ = 64
SLICE_W = 128
N_EXP = 64
NH = 4
L = 16
NC, NS = 2, 16
NW = NC * NS
TPW = N_TOK // NW
GROUPS = TPW // L


def _router_body(x_hbm, seeds_hbm, sel_hbm, mask_hbm,
                 inbuf, xbuf, selbuf, seedbuf, mb0, mb1, sem0, sem1):
    wid = lax.axis_index("s") * jnp.int32(NC) + lax.axis_index("c")
    base = wid * jnp.int32(TPW)

    tb0 = wid * jnp.int32(TPW // 8)
    pltpu.sync_copy(
        x_hbm.at[pl.ds(tb0, TPW // 8), jnp.int32(0), :, pl.ds(0, DIMQ)], inbuf)
    pltpu.sync_copy(seeds_hbm, seedbuf)

    iota = lax.iota(jnp.int32, L)
    zerof = jnp.full((L,), 0.0, jnp.float32)
    sevenf = jnp.full((L,), 7.0, jnp.float32)
    twelve = jnp.full((L,), 12, jnp.int32)
    four = jnp.full((L,), 4, jnp.int32)
    zeroi = jnp.full((L,), 0, jnp.int32)
    half = jnp.full((L,), 0.5, jnp.float32)
    c63 = jnp.full((L,), 63, jnp.int32)
    weights = [iota + jnp.int32(16 * g + 1) for g in range(4)]
    seeds = [seedbuf[h, :] for h in range(NH)]

    mbufs = (mb0, mb1)
    sems = (sem0, sem1)

    def group(it, b):
        g = it * jnp.int32(2) + jnp.int32(b)
        mb = mbufs[b]
        sem = sems[b]
        gl = g * jnp.int32(L)

        @pl.when(it > 0)
        def _wait():
            pltpu.make_async_copy(mb, mask_hbm.at[:, pl.ds(base, L)], sem).wait()

        for r in range(N_EXP):
            mb[r, pl.ds(0, L)] = zerof

        gb = g * jnp.int32(2)
        for i in range(L):
            tb = gb + jnp.int32(i // 8)
            acc = None
            for gg in range(4):
                xg = inbuf[tb, i % 8, pl.ds(16 * gg, 16)]
                mag = jnp.minimum(jnp.abs(xg), sevenf).astype(jnp.int32)
                s2 = jnp.where(xg < zerof, twelve,
                               jnp.where(xg > zerof, four, zeroi))
                term = (s2 | mag) * weights[gg]
                acc = term if acc is None else acc ^ term
            xbuf[i, pl.ds(0, 16)] = acc

        rk = None
        for j in range(L):
            col = plsc.load_gather(xbuf, [iota, jnp.full((L,), j, jnp.int32)])
            rk = col if rk is None else rk ^ col

        e = [(rk ^ seeds[h]) & c63 for h in range(NH)]
        a = jnp.minimum(e[0], e[1])
        bb = jnp.maximum(e[0], e[1])
        c = jnp.minimum(e[2], e[3])
        d = jnp.maximum(e[2], e[3])
        sel0 = jnp.minimum(a, c)
        sel1 = jnp.minimum(jnp.maximum(a, c), jnp.minimum(bb, d))

        selbuf[0, pl.ds(gl, L)] = sel0
        selbuf[1, pl.ds(gl, L)] = sel1

        plsc.store_scatter(mb, [sel0, iota], half)
        plsc.store_scatter(mb, [sel1, iota], half)

        pltpu.async_copy(mb, mask_hbm.at[:, pl.ds(base + gl, L)], sem)

    def body(_, it):
        group(it, 0)
        group(it, 1)
        return it + jnp.int32(1)

    lax.fori_loop(0, GROUPS // 2, body, jnp.int32(0))

    pltpu.make_async_copy(mb0, mask_hbm.at[:, pl.ds(base, L)], sem0).wait()
    pltpu.make_async_copy(mb1, mask_hbm.at[:, pl.ds(base, L)], sem1).wait()
    pltpu.sync_copy(selbuf, sel_hbm.at[:, pl.ds(base, TPW)])


_router = functools.partial(
    pl.kernel,
    out_type=[
        jax.ShapeDtypeStruct((2, N_TOK), jnp.int32),
        jax.ShapeDtypeStruct((N_EXP, N_TOK), jnp.float32),
    ],
    mesh=plsc.VectorSubcoreMesh(core_axis_name="c", subcore_axis_name="s"),
    compiler_params=pltpu.CompilerParams(
        use_tc_tiling_on_sc=False, needs_layout_passes=False),
    scratch_types=[
        pltpu.VMEM((TPW // 8, 8, DIMQ), jnp.float32),
        pltpu.VMEM((L, 17), jnp.int32),
        pltpu.VMEM((2, TPW), jnp.int32),
        pltpu.VMEM((NH, L), jnp.int32),
        pltpu.VMEM((N_EXP, L), jnp.float32),
        pltpu.VMEM((N_EXP, L), jnp.float32),
        pltpu.SemaphoreType.DMA,
        pltpu.SemaphoreType.DMA,
    ],
)(_router_body)


@jax.jit
def kernel(hidden_states, hash_seeds):
    xs = hidden_states.reshape(
        N_TOK // 8, 8, HID // 128, 128).transpose(0, 2, 1, 3)
    seeds_b = jnp.broadcast_to(
        hash_seeds.astype(jnp.int32)[:, None], (NH, L))
    sel_planar, masks_t = _router(xs, seeds_b)
    masks = masks_t.T
    selected = sel_planar.astype(jnp.int64).T
    expert_weights = jnp.full((N_TOK, 2), 0.5, dtype=jnp.float32)
    return selected, expert_weights, masks


# speedup vs baseline: 106.6636x; 1.1875x over previous
import functools

import jax
import jax.numpy as jnp
from jax import lax
from jax.experimental import pallas as pl
from jax.experimental.pallas import tpu as pltpu
from jax.experimental.pallas import tpu_sc as plsc

N_TOK = 32768
HID = 1024
DIMQ = 64
SLICE_W = 128
N_EXP = 64
NH = 4
L = 16
NC, NS = 2, 16
NW = NC * NS
TPW = N_TOK // NW
GROUPS = TPW // L


def _router_body(x_hbm, seeds_hbm, sel_hbm, mask_hbm,
                 inbuf, xbuf, selbuf, seedbuf, mb0, mb1, sem0, sem1):
    wid = lax.axis_index("s") * jnp.int32(NC) + lax.axis_index("c")
    base = wid * jnp.int32(TPW)

    tb0 = wid * jnp.int32(TPW // 8)
    pltpu.sync_copy(
        x_hbm.at[pl.ds(tb0, TPW // 8), jnp.int32(0), :, pl.ds(0, DIMQ)], inbuf)
    pltpu.sync_copy(seeds_hbm, seedbuf)

    iota = lax.iota(jnp.int32, L)
    zerof = jnp.full((L,), 0.0, jnp.float32)
    sevenf = jnp.full((L,), 7.0, jnp.float32)
    twelve = jnp.full((L,), 12, jnp.int32)
    four = jnp.full((L,), 4, jnp.int32)
    zeroi = jnp.full((L,), 0, jnp.int32)
    half = jnp.full((L,), 0.5, jnp.float32)
    c63 = jnp.full((L,), 63, jnp.int32)
    weights = [iota + jnp.int32(16 * g + 1) for g in range(4)]
    seeds = [seedbuf[h, :] for h in range(NH)]

    mbufs = (mb0, mb1)
    sems = (sem0, sem1)

    def group(it, b):
        g = it * jnp.int32(2) + jnp.int32(b)
        mb = mbufs[b]
        sem = sems[b]
        gl = g * jnp.int32(L)

        @pl.when(it > 0)
        def _wait():
            pltpu.make_async_copy(
                mb, mask_hbm.at[:, jnp.int32(0), :, pl.ds(0, L)], sem).wait()

        for rb in range(N_EXP // 8):
            for r in range(8):
                mb[rb, r, pl.ds(0, L)] = zerof

        gb = g * jnp.int32(2)
        for i in range(L):
            tb = gb + jnp.int32(i // 8)
            acc = None
            for gg in range(4):
                xg = inbuf[tb, i % 8, pl.ds(16 * gg, 16)]
                mag = jnp.minimum(jnp.abs(xg), sevenf).astype(jnp.int32)
                s2 = jnp.where(xg < zerof, twelve,
                               jnp.where(xg > zerof, four, zeroi))
                term = (s2 | mag) * weights[gg]
                acc = term if acc is None else acc ^ term
            xbuf[i, pl.ds(0, 16)] = acc

        rk = None
        for j in range(L):
            col = plsc.load_gather(xbuf, [iota, jnp.full((L,), j, jnp.int32)])
            rk = col if rk is None else rk ^ col

        e = [(rk ^ seeds[h]) & c63 for h in range(NH)]
        a = jnp.minimum(e[0], e[1])
        bb = jnp.maximum(e[0], e[1])
        c = jnp.minimum(e[2], e[3])
        d = jnp.maximum(e[2], e[3])
        sel0 = jnp.minimum(a, c)
        sel1 = jnp.minimum(jnp.maximum(a, c), jnp.minimum(bb, d))

        selbuf[0, pl.ds(gl, L)] = sel0
        selbuf[1, pl.ds(gl, L)] = sel1

        c7 = jnp.full((L,), 7, jnp.int32)
        plsc.store_scatter(mb, [sel0 >> jnp.int32(3), sel0 & c7, iota], half)
        plsc.store_scatter(mb, [sel1 >> jnp.int32(3), sel1 & c7, iota], half)

        tglob = base + gl
        cbi = tglob >> jnp.int32(7)
        c0 = pl.multiple_of(tglob & jnp.int32(127), L)
        pltpu.async_copy(mb, mask_hbm.at[:, cbi, :, pl.ds(c0, L)], sem)

    def body(_, it):
        group(it, 0)
        group(it, 1)
        return it + jnp.int32(1)

    lax.fori_loop(0, GROUPS // 2, body, jnp.int32(0))

    pltpu.make_async_copy(
        mb0, mask_hbm.at[:, jnp.int32(0), :, pl.ds(0, L)], sem0).wait()
    pltpu.make_async_copy(
        mb1, mask_hbm.at[:, jnp.int32(0), :, pl.ds(0, L)], sem1).wait()
    pltpu.sync_copy(selbuf, sel_hbm.at[:, pl.ds(base, TPW)])


_router = functools.partial(
    pl.kernel,
    out_type=[
        jax.ShapeDtypeStruct((2, N_TOK), jnp.int32),
        jax.ShapeDtypeStruct((N_EXP // 8, N_TOK // 128, 8, 128), jnp.float32),
    ],
    mesh=plsc.VectorSubcoreMesh(core_axis_name="c", subcore_axis_name="s"),
    compiler_params=pltpu.CompilerParams(
        use_tc_tiling_on_sc=False, needs_layout_passes=False),
    scratch_types=[
        pltpu.VMEM((TPW // 8, 8, DIMQ), jnp.float32),
        pltpu.VMEM((L, 17), jnp.int32),
        pltpu.VMEM((2, TPW), jnp.int32),
        pltpu.VMEM((NH, L), jnp.int32),
        pltpu.VMEM((N_EXP // 8, 8, L), jnp.float32),
        pltpu.VMEM((N_EXP // 8, 8, L), jnp.float32),
        pltpu.SemaphoreType.DMA,
        pltpu.SemaphoreType.DMA,
    ],
)(_router_body)


@jax.jit
def kernel(hidden_states, hash_seeds):
    xs = hidden_states.reshape(
        N_TOK // 8, 8, HID // 128, 128).transpose(0, 2, 1, 3)
    seeds_b = jnp.broadcast_to(
        hash_seeds.astype(jnp.int32)[:, None], (NH, L))
    sel_planar, masks4 = _router(xs, seeds_b)
    masks = masks4.transpose(1, 3, 0, 2).reshape(N_TOK, N_EXP)
    selected = sel_planar.astype(jnp.int64).T
    expert_weights = jnp.full((N_TOK, 2), 0.5, dtype=jnp.float32)
    return selected, expert_weights, masks


# speedup vs baseline: 111.5187x; 1.0455x over previous
import functools

import jax
import jax.numpy as jnp
from jax import lax
from jax.experimental import pallas as pl
from jax.experimental.pallas import tpu as pltpu
from jax.experimental.pallas import tpu_sc as plsc

N_TOK = 32768
HID = 1024
DIMQ = 64
SLICE_W = 128
N_EXP = 64
NH = 4
L = 16
NC, NS = 2, 16
NW = NC * NS
TPW = N_TOK // NW
GROUPS = TPW // L


def _router_body(x_hbm, seeds_hbm, sel_hbm, mask_hbm,
                 inbuf, xbuf, selbuf, seedbuf, mb0, mb1, sem0, sem1,
                 semq0, semq1, semq2, semq3):
    wid = lax.axis_index("s") * jnp.int32(NC) + lax.axis_index("c")
    base = wid * jnp.int32(TPW)

    tb0 = wid * jnp.int32(TPW // 8)
    semq = (semq0, semq1, semq2, semq3)
    QB = TPW // 8 // 4

    def chunk_copy(q):
        return pltpu.make_async_copy(
            x_hbm.at[pl.ds(tb0 + jnp.int32(q * QB), QB),
                     jnp.int32(0), :, pl.ds(0, DIMQ)],
            inbuf.at[pl.ds(q * QB, QB)], semq[q])

    for q in range(4):
        chunk_copy(q).start()
    pltpu.sync_copy(seeds_hbm, seedbuf)
    chunk_copy(0).wait()

    iota = lax.iota(jnp.int32, L)
    zerof = jnp.full((L,), 0.0, jnp.float32)
    sevenf = jnp.full((L,), 7.0, jnp.float32)
    twelve = jnp.full((L,), 12, jnp.int32)
    four = jnp.full((L,), 4, jnp.int32)
    zeroi = jnp.full((L,), 0, jnp.int32)
    half = jnp.full((L,), 0.5, jnp.float32)
    c63 = jnp.full((L,), 63, jnp.int32)
    weights = [iota + jnp.int32(16 * g + 1) for g in range(4)]
    seeds = [seedbuf[h, :] for h in range(NH)]

    mbufs = (mb0, mb1)
    sems = (sem0, sem1)

    c7 = jnp.full((L,), 7, jnp.int32)

    for mb in mbufs:
        for rb in range(N_EXP // 8):
            for r in range(8):
                mb[rb, r, pl.ds(0, L)] = zerof

    def group(it, b, p0, p1):
        g = it * jnp.int32(2) + jnp.int32(b)
        mb = mbufs[b]
        sem = sems[b]
        gl = g * jnp.int32(L)

        @pl.when(it > 0)
        def _wait():
            pltpu.make_async_copy(
                mb, mask_hbm.at[:, jnp.int32(0), :, pl.ds(0, L)], sem).wait()

        plsc.store_scatter(mb, [p0 >> jnp.int32(3), p0 & c7, iota], zerof)
        plsc.store_scatter(mb, [p1 >> jnp.int32(3), p1 & c7, iota], zerof)

        gb = g * jnp.int32(2)
        for i in range(L):
            tb = gb + jnp.int32(i // 8)
            acc = None
            for gg in range(4):
                xg = inbuf[tb, i % 8, pl.ds(16 * gg, 16)]
                mag = jnp.minimum(jnp.abs(xg), sevenf).astype(jnp.int32)
                s2 = jnp.where(xg < zerof, twelve,
                               jnp.where(xg > zerof, four, zeroi))
                term = (s2 | mag) * weights[gg]
                acc = term if acc is None else acc ^ term
            xbuf[i, pl.ds(0, 16)] = acc

        cols = [plsc.load_gather(xbuf, [iota, jnp.full((L,), j, jnp.int32)])
                for j in range(L)]
        while len(cols) > 1:
            cols = [cols[2 * k] ^ cols[2 * k + 1] for k in range(len(cols) // 2)]
        rk = cols[0]

        e = [(rk ^ seeds[h]) & c63 for h in range(NH)]
        a = jnp.minimum(e[0], e[1])
        bb = jnp.maximum(e[0], e[1])
        c = jnp.minimum(e[2], e[3])
        d = jnp.maximum(e[2], e[3])
        sel0 = jnp.minimum(a, c)
        sel1 = jnp.minimum(jnp.maximum(a, c), jnp.minimum(bb, d))

        selbuf[0, pl.ds(gl, L)] = sel0
        selbuf[1, pl.ds(gl, L)] = sel1

        plsc.store_scatter(mb, [sel0 >> jnp.int32(3), sel0 & c7, iota], half)
        plsc.store_scatter(mb, [sel1 >> jnp.int32(3), sel1 & c7, iota], half)

        tglob = base + gl
        cbi = tglob >> jnp.int32(7)
        c0 = pl.multiple_of(tglob & jnp.int32(127), L)
        pltpu.async_copy(mb, mask_hbm.at[:, cbi, :, pl.ds(c0, L)], sem)
        return sel0, sel1

    def body(_, carry):
        it, pa0, pa1, pb0, pb1 = carry
        for q in range(1, 4):
            @pl.when(it == 8 * q)
            def _wq(q=q):
                chunk_copy(q).wait()
        sa0, sa1 = group(it, 0, pa0, pa1)
        sb0, sb1 = group(it, 1, pb0, pb1)
        return it + jnp.int32(1), sa0, sa1, sb0, sb1

    lax.fori_loop(0, GROUPS // 2, body,
                  (jnp.int32(0), zeroi, zeroi, zeroi, zeroi))

    pltpu.make_async_copy(
        mb0, mask_hbm.at[:, jnp.int32(0), :, pl.ds(0, L)], sem0).wait()
    pltpu.make_async_copy(
        mb1, mask_hbm.at[:, jnp.int32(0), :, pl.ds(0, L)], sem1).wait()
    pltpu.sync_copy(selbuf, sel_hbm.at[:, pl.ds(base, TPW)])


_router = functools.partial(
    pl.kernel,
    out_type=[
        jax.ShapeDtypeStruct((2, N_TOK), jnp.int32),
        jax.ShapeDtypeStruct((N_EXP // 8, N_TOK // 128, 8, 128), jnp.float32),
    ],
    mesh=plsc.VectorSubcoreMesh(core_axis_name="c", subcore_axis_name="s"),
    compiler_params=pltpu.CompilerParams(
        use_tc_tiling_on_sc=False, needs_layout_passes=False),
    scratch_types=[
        pltpu.VMEM((TPW // 8, 8, DIMQ), jnp.float32),
        pltpu.VMEM((L, 17), jnp.int32),
        pltpu.VMEM((2, TPW), jnp.int32),
        pltpu.VMEM((NH, L), jnp.int32),
        pltpu.VMEM((N_EXP // 8, 8, L), jnp.float32),
        pltpu.VMEM((N_EXP // 8, 8, L), jnp.float32),
        pltpu.SemaphoreType.DMA,
        pltpu.SemaphoreType.DMA,
        pltpu.SemaphoreType.DMA,
        pltpu.SemaphoreType.DMA,
        pltpu.SemaphoreType.DMA,
        pltpu.SemaphoreType.DMA,
    ],
)(_router_body)


@jax.jit
def kernel(hidden_states, hash_seeds):
    xs = hidden_states.reshape(
        N_TOK // 8, 8, HID // 128, 128).transpose(0, 2, 1, 3)
    seeds_b = jnp.broadcast_to(
        hash_seeds.astype(jnp.int32)[:, None], (NH, L))
    sel_planar, masks4 = _router(xs, seeds_b)
    masks = masks4.transpose(1, 3, 0, 2).reshape(N_TOK, N_EXP)
    selected = sel_planar.astype(jnp.int64).T
    expert_weights = jnp.full((N_TOK, 2), 0.5, dtype=jnp.float32)
    return selected, expert_weights, masks
